# baseline (speedup 1.0000x reference)
import jax
import jax.numpy as jnp
from jax.experimental import pallas as pl

P = 0.5
ROWS = 64
COLS = 160000
COL_BLOCK = 16000


def _row_signs():
    mask = jax.random.uniform(jax.random.key(42), (ROWS,)) < P
    return jnp.where(mask, -1.0, 1.0).astype(jnp.float32)


def _scale_body(s_ref, x_ref, o_ref):
    o_ref[...] = x_ref[...] * s_ref[...]


def kernel(x):
    signs = _row_signs().reshape(ROWS, 1)
    x2 = x.reshape(ROWS, COLS)
    out = pl.pallas_call(
        _scale_body,
        grid=(COLS // COL_BLOCK,),
        in_specs=[
            pl.BlockSpec((ROWS, 1), lambda i: (0, 0)),
            pl.BlockSpec((ROWS, COL_BLOCK), lambda i: (0, i)),
        ],
        out_specs=pl.BlockSpec((ROWS, COL_BLOCK), lambda i: (0, i)),
        out_shape=jax.ShapeDtypeStruct((ROWS, COLS), jnp.float32),
    )(signs, x2)
    return out.reshape(x.shape)
